# Initial kernel scaffold; baseline (speedup 1.0000x reference)
import functools

import jax
import jax.numpy as jnp
from jax import lax
from jax.experimental import pallas as pl
from jax.experimental.pallas import tpu as pltpu
from jax.experimental.pallas import tpu_sc as plsc

_N = 10000
_E = 320000
_NC = 2
_NS = 16
_NW = _NC * _NS
_CW = 125
_CH = _E // (_NW * _CW)
_RPT = _N // _NS


def _sc_segsum(table, sidx3, didx3, d):
    mesh = plsc.VectorSubcoreMesh(core_axis_name="c", subcore_axis_name="s")

    @functools.partial(
        pl.kernel,
        mesh=mesh,
        out_type=jax.ShapeDtypeStruct((_NC, _N, d), jnp.float32),
        scratch_types=[
            pltpu.VMEM((_CH, _CW), jnp.int32),
            pltpu.VMEM((_CH, _CW), jnp.int32),
            pltpu.VMEM((_CW, d), jnp.float32),
            pltpu.VMEM((_CW, d), jnp.float32),
            pltpu.VMEM_SHARED((_N, d), jnp.float32),
            pltpu.SemaphoreType.DMA,
            pltpu.SemaphoreType.DMA,
        ],
    )
    def k(table_h, sidx_h, didx_h, out_h, sidx_v, didx_v, msg0, msg1, acc,
          sem0, sem1):
        c = lax.axis_index("c")
        s = lax.axis_index("s")
        wid = c * _NS + s

        @pl.loop(0, _CW)
        def _(i):
            @pl.loop(0, d // 16)
            def _(kk):
                msg0[i, pl.ds(kk * 16, 16)] = jnp.zeros((16,), jnp.float32)

        @pl.loop(0, _RPT // _CW)
        def _(kk):
            pltpu.sync_copy(msg0, acc.at[pl.ds(s * _RPT + kk * _CW, _CW)])

        pltpu.sync_copy(sidx_h.at[wid], sidx_v)
        pltpu.sync_copy(didx_h.at[wid], didx_v)
        plsc.subcore_barrier()

        pltpu.make_async_copy(table_h.at[sidx_v.at[0]], msg0, sem0).start()

        @pl.loop(0, _CH, step=2)
        def _(j):
            pltpu.make_async_copy(table_h.at[sidx_v.at[j + 1]], msg1,
                                  sem1).start()
            pltpu.make_async_copy(table_h.at[sidx_v.at[j]], msg0, sem0).wait()
            pltpu.sync_copy(msg0, acc.at[didx_v.at[j]], add=True)

            @pl.when(j + 2 < _CH)
            def _():
                pltpu.make_async_copy(table_h.at[sidx_v.at[j + 2]], msg0,
                                      sem0).start()

            pltpu.make_async_copy(table_h.at[sidx_v.at[j + 1]], msg1,
                                  sem1).wait()
            pltpu.sync_copy(msg1, acc.at[didx_v.at[j + 1]], add=True)

        plsc.subcore_barrier()
        pltpu.sync_copy(acc.at[pl.ds(s * _RPT, _RPT)],
                        out_h.at[c, pl.ds(s * _RPT, _RPT)])

    return k(table, sidx3, didx3)


def _dense1(p_ref, norm_ref, w1_ref, b1_ref, w2a_ref, w2b_ref, o_ref):
    agg = p_ref[0] + p_ref[1]
    hs = agg * norm_ref[...]
    h = lax.dot_general(hs, w1_ref[...], (((1,), (0,)), ((), ())),
                        precision=lax.Precision.HIGHEST,
                        preferred_element_type=jnp.float32) + b1_ref[...]
    hr = jnp.maximum(h, 0.0)
    o_ref[...] = (
        lax.dot_general(h, w2a_ref[...], (((1,), (0,)), ((), ())),
                        precision=lax.Precision.HIGHEST,
                        preferred_element_type=jnp.float32)
        + lax.dot_general(hr, w2b_ref[...], (((1,), (0,)), ((), ())),
                          precision=lax.Precision.HIGHEST,
                          preferred_element_type=jnp.float32))


def _dense2(q_ref, norm_ref, b2_ref, o_ref):
    o_ref[...] = (q_ref[0] + q_ref[1]) * norm_ref[...] + b2_ref[...]


def kernel(x, edge_index, norm, W1, b1, W2, b2):
    src3 = edge_index[0].reshape(_NW, _CH, _CW)
    dst3 = edge_index[1].reshape(_NW, _CH, _CW)
    b1r = b1.reshape(1, -1)
    b2r = b2.reshape(1, -1)
    W2a = W2[:128]
    W2b = W2[128:]

    part1 = _sc_segsum(x, src3, dst3, 128)

    B1 = 1000
    p = pl.pallas_call(
        _dense1,
        grid=(_N // B1,),
        in_specs=[
            pl.BlockSpec((_NC, B1, 128), lambda i: (0, i, 0)),
            pl.BlockSpec((B1, 1), lambda i: (i, 0)),
            pl.BlockSpec((128, 128), lambda i: (0, 0)),
            pl.BlockSpec((1, 128), lambda i: (0, 0)),
            pl.BlockSpec((128, 64), lambda i: (0, 0)),
            pl.BlockSpec((128, 64), lambda i: (0, 0)),
        ],
        out_specs=pl.BlockSpec((B1, 64), lambda i: (i, 0)),
        out_shape=jax.ShapeDtypeStruct((_N, 64), jnp.float32),
    )(part1, norm, W1, b1r, W2a, W2b)

    part2 = _sc_segsum(p, src3, dst3, 64)

    B2 = 2000
    out = pl.pallas_call(
        _dense2,
        grid=(_N // B2,),
        in_specs=[
            pl.BlockSpec((_NC, B2, 64), lambda i: (0, i, 0)),
            pl.BlockSpec((B2, 1), lambda i: (i, 0)),
            pl.BlockSpec((1, 64), lambda i: (0, 0)),
        ],
        out_specs=pl.BlockSpec((B2, 64), lambda i: (i, 0)),
        out_shape=jax.ShapeDtypeStruct((_N, 64), jnp.float32),
    )(part2, norm, b2r)

    return out


# speedup vs baseline: 11.6982x; 11.6982x over previous
import functools

import jax
import jax.numpy as jnp
from jax import lax
from jax.experimental import pallas as pl
from jax.experimental.pallas import tpu as pltpu
from jax.experimental.pallas import tpu_sc as plsc

_N = 10000
_E = 320000
_D = 64
_NC = 2
_NS = 16
_NW = _NC * _NS
_CW = 125
_CH = _E // (_NW * _CW)
_NP = 10240
_RPT = _NP // _NS
_ZB = 128

_MESH = plsc.VectorSubcoreMesh(core_axis_name="c", subcore_axis_name="s")


@functools.partial(
    pl.kernel,
    mesh=_MESH,
    out_type=jax.ShapeDtypeStruct((_NC, _NP, _D), jnp.float32),
    scratch_types=[
        pltpu.VMEM((_CH, _CW), jnp.int32),
        pltpu.VMEM((_CH, _CW), jnp.int32),
        pltpu.VMEM((_CW, _D), jnp.float32),
        pltpu.VMEM((_CW, _D), jnp.float32),
        pltpu.VMEM((_ZB, _D), jnp.float32),
        pltpu.VMEM_SHARED((_NP, _D), jnp.float32),
        pltpu.SemaphoreType.DMA,
        pltpu.SemaphoreType.DMA,
    ],
    compiler_params=pltpu.CompilerParams(use_tc_tiling_on_sc=False),
)
def _sc_segsum(table_h, sidx_h, didx_h, out_h, sidx_v, didx_v, msg0, msg1,
               zbuf, acc, sem0, sem1):
    c = lax.axis_index("c")
    s = lax.axis_index("s")
    wid = c * _NS + s

    @pl.loop(0, _ZB)
    def _(i):
        @pl.loop(0, _D // 16)
        def _(kk):
            zbuf[i, pl.ds(kk * 16, 16)] = jnp.zeros((16,), jnp.float32)

    @pl.loop(0, _RPT // _ZB)
    def _(kk):
        pltpu.sync_copy(zbuf, acc.at[pl.ds(s * _RPT + kk * _ZB, _ZB)])

    pltpu.sync_copy(sidx_h.at[wid], sidx_v)
    pltpu.sync_copy(didx_h.at[wid], didx_v)
    plsc.subcore_barrier()

    pltpu.make_async_copy(table_h.at[sidx_v.at[0]], msg0, sem0).start()

    @pl.loop(0, _CH, step=2)
    def _(j):
        pltpu.make_async_copy(table_h.at[sidx_v.at[j + 1]], msg1,
                              sem1).start()
        pltpu.make_async_copy(table_h.at[sidx_v.at[j]], msg0, sem0).wait()
        pltpu.sync_copy(msg0, acc.at[didx_v.at[j]], add=True)

        @pl.when(j + 2 < _CH)
        def _():
            pltpu.make_async_copy(table_h.at[sidx_v.at[j + 2]], msg0,
                                  sem0).start()

        pltpu.make_async_copy(table_h.at[sidx_v.at[j + 1]], msg1,
                              sem1).wait()
        pltpu.sync_copy(msg1, acc.at[didx_v.at[j + 1]], add=True)

    plsc.subcore_barrier()
    pltpu.sync_copy(acc.at[pl.ds(s * _RPT, _RPT)],
                    out_h.at[c, pl.ds(s * _RPT, _RPT)])


def _dense1(pa_ref, pb_ref, norm_ref, w1a_ref, w1b_ref, b1_ref, w2a_ref,
            w2b_ref, o_ref):
    nrm = norm_ref[...]
    hsa = (pa_ref[0] + pa_ref[1]) * nrm
    hsb = (pb_ref[0] + pb_ref[1]) * nrm
    dn = (((1,), (0,)), ((), ()))
    h = (lax.dot_general(hsa, w1a_ref[...], dn,
                         precision=lax.Precision.HIGHEST,
                         preferred_element_type=jnp.float32)
         + lax.dot_general(hsb, w1b_ref[...], dn,
                           precision=lax.Precision.HIGHEST,
                           preferred_element_type=jnp.float32)
         + b1_ref[...])
    hr = jnp.maximum(h, 0.0)
    o_ref[...] = (
        lax.dot_general(h, w2a_ref[...], dn,
                        precision=lax.Precision.HIGHEST,
                        preferred_element_type=jnp.float32)
        + lax.dot_general(hr, w2b_ref[...], dn,
                          precision=lax.Precision.HIGHEST,
                          preferred_element_type=jnp.float32))


def _dense2(q_ref, norm_ref, b2_ref, o_ref):
    o_ref[...] = (q_ref[0] + q_ref[1]) * norm_ref[...] + b2_ref[...]


def kernel(x, edge_index, norm, W1, b1, W2, b2):
    src3 = edge_index[0].reshape(_NW, _CH, _CW)
    dst3 = edge_index[1].reshape(_NW, _CH, _CW)
    xa = x[:, :64]
    xb = x[:, 64:]
    b1r = b1.reshape(1, -1)
    b2r = b2.reshape(1, -1)
    W1a = W1[:64]
    W1b = W1[64:]
    W2a = W2[:128]
    W2b = W2[128:]

    pa = _sc_segsum(xa, src3, dst3)
    pb = _sc_segsum(xb, src3, dst3)

    B1 = 1000
    p = pl.pallas_call(
        _dense1,
        grid=(_N // B1,),
        in_specs=[
            pl.BlockSpec((_NC, B1, 64), lambda i: (0, i, 0)),
            pl.BlockSpec((_NC, B1, 64), lambda i: (0, i, 0)),
            pl.BlockSpec((B1, 1), lambda i: (i, 0)),
            pl.BlockSpec((64, 128), lambda i: (0, 0)),
            pl.BlockSpec((64, 128), lambda i: (0, 0)),
            pl.BlockSpec((1, 128), lambda i: (0, 0)),
            pl.BlockSpec((128, 64), lambda i: (0, 0)),
            pl.BlockSpec((128, 64), lambda i: (0, 0)),
        ],
        out_specs=pl.BlockSpec((B1, 64), lambda i: (i, 0)),
        out_shape=jax.ShapeDtypeStruct((_N, 64), jnp.float32),
    )(pa, pb, norm, W1a, W1b, b1r, W2a, W2b)

    part2 = _sc_segsum(p, src3, dst3)

    B2 = 2000
    out = pl.pallas_call(
        _dense2,
        grid=(_N // B2,),
        in_specs=[
            pl.BlockSpec((_NC, B2, 64), lambda i: (0, i, 0)),
            pl.BlockSpec((B2, 1), lambda i: (i, 0)),
            pl.BlockSpec((1, 64), lambda i: (0, 0)),
        ],
        out_specs=pl.BlockSpec((B2, 64), lambda i: (i, 0)),
        out_shape=jax.ShapeDtypeStruct((_N, 64), jnp.float32),
    )(part2, norm, b2r)

    return out
